# Initial kernel scaffold; baseline (speedup 1.0000x reference)
import jax
import jax.numpy as jnp
from jax.experimental import pallas as pl


def kernel(x, W):
    raise NotImplementedError("write your pallas kernel here")


# speedup vs baseline: 1.0274x; 1.0274x over previous
import functools

import jax
import jax.numpy as jnp
from jax import lax
from jax.experimental import pallas as pl
from jax.experimental.pallas import tpu as pltpu
from jax.experimental.pallas import tpu_sc as plsc

_D = 256
_K = 8192
_B = 8192
_BN = 512
_BK = 2048

_NC, _NS = 2, 16
_NW = _NC * _NS
_BPW = _B // _NW


def _dist_body(z_ref, wt_ref, z2_ref, w2_ref, idx_ref, dmin_ref):
    z = z_ref[...]
    z2 = z2_ref[...]
    nj = _K // _BK

    def body(j, carry):
        run_val, run_idx, run_true = carry
        wt = wt_ref[:, pl.ds(j * _BK, _BK)]
        t = jnp.dot(z, wt, preferred_element_type=jnp.float32)
        w2 = w2_ref[:, pl.ds(j * _BK, _BK)]
        d = (z2 - 2.0 * t) + w2
        lv = jnp.min(d, axis=1)
        ii = lax.broadcasted_iota(jnp.int32, d.shape, 1)
        li = jnp.min(jnp.where(d == lv[:, None], ii, _K), axis=1) + (j * _BK)
        better = lv < run_val
        lq = lv.astype(jnp.bfloat16).astype(jnp.float32)
        return (jnp.where(better, lq, run_val),
                jnp.where(better, li, run_idx),
                jnp.where(better, lv, run_true))

    rv, ri, rt = lax.fori_loop(
        0, nj, body,
        (jnp.full((_BN,), jnp.inf, jnp.float32),
         jnp.zeros((_BN,), jnp.int32),
         jnp.full((_BN,), jnp.inf, jnp.float32)))
    idx_ref[...] = ri[:, None]
    dmin_ref[...] = rt[:, None]


def _nearest(z, wt, z2, w2):
    n = z.shape[0]
    return pl.pallas_call(
        _dist_body,
        grid=(n // _BN,),
        in_specs=[
            pl.BlockSpec((_BN, _D), lambda i: (i, 0)),
            pl.BlockSpec((_D, _K), lambda i: (0, 0)),
            pl.BlockSpec((_BN, 1), lambda i: (i, 0)),
            pl.BlockSpec((1, _K), lambda i: (0, 0)),
        ],
        out_specs=[
            pl.BlockSpec((_BN, 1), lambda i: (i, 0)),
            pl.BlockSpec((_BN, 1), lambda i: (i, 0)),
        ],
        out_shape=[
            jax.ShapeDtypeStruct((n, 1), jnp.int32),
            jax.ShapeDtypeStruct((n, 1), jnp.float32),
        ],
    )(z, wt, z2, w2)


def _gather_body(table_hbm, idx_hbm, out_hbm, idx_v, rows_v, sem):
    wid = lax.axis_index("s") * _NC + lax.axis_index("c")
    base = wid * _BPW
    pltpu.sync_copy(idx_hbm.at[pl.ds(base, _BPW)], idx_v)
    pltpu.async_copy(table_hbm.at[idx_v], rows_v, sem).wait()
    pltpu.sync_copy(rows_v, out_hbm.at[pl.ds(base, _BPW)])


def _gather(W, idx):
    k = functools.partial(
        pl.kernel,
        out_type=jax.ShapeDtypeStruct((_B, _D), jnp.float32),
        mesh=plsc.VectorSubcoreMesh(core_axis_name="c", subcore_axis_name="s"),
        scratch_types=[
            pltpu.VMEM((_BPW,), jnp.int32),
            pltpu.VMEM((_BPW, _D), jnp.float32),
            pltpu.SemaphoreType.DMA,
        ],
    )(_gather_body)
    return k(W, idx)


def kernel(x, W):
    z = x.reshape(-1, _D)
    z2 = jnp.sum(z * z, axis=1, keepdims=True)
    w2 = jnp.sum(W * W, axis=1)[None, :]
    idx2, dmin2 = _nearest(z.astype(jnp.bfloat16),
                           W.T.astype(jnp.bfloat16), z2, w2)
    idx = idx2[:, 0]
    zq = _gather(W, idx)
    loss = jnp.sum(dmin2[:, 0]) / jnp.float32(z.size)
    return (zq.reshape(x.shape), loss, loss,
            idx.reshape(x.shape[:-1]))
